# Initial kernel scaffold; baseline (speedup 1.0000x reference)
import jax
import jax.numpy as jnp
from jax.experimental import pallas as pl


def kernel(entity_embedding, entity_embedding_bias, bases1, comb1, b1, bases2, comb2, b2, rel_emb, edge_index, edge_type):
    raise NotImplementedError("write your pallas kernel here")


# speedup vs baseline: 4.1015x; 4.1015x over previous
import functools

import jax
import jax.numpy as jnp
from jax import lax
from jax.experimental import pallas as pl
from jax.experimental.pallas import tpu as pltpu
from jax.experimental.pallas import tpu_sc as plsc

N = 10000
R = 50
NUM_REL = 2 * R + 1
D = 128
H = 128
B = 8
E = 160000
E_F = 2 * E + N

NC = 2
NS = 16
NW = NC * NS
C = 384
NCHUNK = 28
E_PAD = NW * C * NCHUNK

ROWS_PAD = 10240
RPT = ROWS_PAD // NW
C1 = 2048

_MESH = plsc.VectorSubcoreMesh(core_axis_name="c", subcore_axis_name="s",
                               num_cores=NC, num_subcores=NS)


def _wid():
    return lax.axis_index("c") * NS + lax.axis_index("s")


@functools.partial(
    pl.kernel,
    out_type=jax.ShapeDtypeStruct((ROWS_PAD * 128,), jnp.float32),
    mesh=_MESH,
    compiler_params=pltpu.CompilerParams(needs_layout_passes=False),
    scratch_types=[
        pltpu.VMEM((C1,), jnp.int32),
        pltpu.VMEM((RPT * 128,), jnp.float32),
    ],
)
def _count_sc(key_hbm, cnt_hbm, key_v, cnt_t):
    wid = _wid()
    key_lo = wid * RPT * 128

    def _zero(i, _):
        cnt_t[pl.ds(i * 16, 16)] = jnp.zeros((16,), jnp.float32)
        return 0
    lax.fori_loop(0, RPT * 8, _zero, 0)

    ones = jnp.ones((16,), jnp.float32)
    def _chunk(ch, _):
        pltpu.sync_copy(key_hbm.at[pl.ds(ch * C1, C1)], key_v)
        def _grp(g, _):
            kv = key_v[pl.ds(g * 16, 16)]
            lkey = kv - key_lo
            msk = (kv >= key_lo) & (kv < key_lo + RPT * 128)
            plsc.addupdate_scatter(cnt_t, [lkey], ones, mask=msk)
            return 0
        lax.fori_loop(0, C1 // 16, _grp, 0)
        return 0
    lax.fori_loop(0, E_PAD // C1, _chunk, 0)

    pltpu.sync_copy(cnt_t, cnt_hbm.at[pl.ds(key_lo, RPT * 128)])


@functools.partial(
    pl.kernel,
    out_type=(jax.ShapeDtypeStruct((E_PAD, D), jnp.float32),
              jax.ShapeDtypeStruct((E_PAD,), jnp.float32)),
    mesh=_MESH,
    compiler_params=pltpu.CompilerParams(needs_layout_passes=False),
    scratch_types=[
        pltpu.VMEM((C,), jnp.int32),
        pltpu.VMEM((C,), jnp.int32),
        pltpu.VMEM((C,), jnp.int32),
        pltpu.VMEM((C, D), jnp.float32),
        pltpu.VMEM((C, 128), jnp.float32),
        pltpu.VMEM((C,), jnp.float32),
        pltpu.SemaphoreType.DMA,
        pltpu.SemaphoreType.DMA,
    ],
)
def _gather_norm_sc(tab_hbm, cnt_hbm, src_hbm, dst_hbm, rel_hbm,
                    out_hbm, norm_hbm,
                    idx_v, dst_v, rel_v, row_v, crow_v, nrm_v, sem1, sem2):
    base = _wid() * C * NCHUNK
    lanes = lax.iota(jnp.int32, 16)
    def _body(i, _):
        off = pl.multiple_of(base + i * C, C)
        pltpu.sync_copy(src_hbm.at[pl.ds(off, C)], idx_v)
        cp1 = pltpu.async_copy(tab_hbm.at[idx_v], row_v, sem1)
        pltpu.sync_copy(dst_hbm.at[pl.ds(off, C)], dst_v)
        cp2 = pltpu.async_copy(cnt_hbm.at[dst_v], crow_v, sem2)
        pltpu.sync_copy(rel_hbm.at[pl.ds(off, C)], rel_v)
        cp1.wait()
        pltpu.sync_copy(row_v, out_hbm.at[pl.ds(off, C)])
        cp2.wait()
        def _grp(g, _):
            rows = g * 16 + lanes
            lane = rel_v[pl.ds(g * 16, 16)]
            cv = plsc.load_gather(crow_v, [rows, lane])
            nrm_v[pl.ds(g * 16, 16)] = 1.0 / jnp.maximum(cv, 1.0)
            return 0
        lax.fori_loop(0, C // 16, _grp, 0)
        pltpu.sync_copy(nrm_v, norm_hbm.at[pl.ds(off, C)])
        return 0
    lax.fori_loop(0, NCHUNK, _body, 0)


@functools.partial(
    pl.kernel,
    out_type=jax.ShapeDtypeStruct((E_PAD, D), jnp.float32),
    mesh=_MESH,
    compiler_params=pltpu.CompilerParams(needs_layout_passes=False),
    scratch_types=[
        pltpu.VMEM((C,), jnp.int32),
        pltpu.VMEM((C, D), jnp.float32),
        pltpu.SemaphoreType.DMA,
    ],
)
def _gather_sc(tab_hbm, idx_hbm, out_hbm, idx_v, row_v, sem):
    base = _wid() * C * NCHUNK
    def _body(i, _):
        off = pl.multiple_of(base + i * C, C)
        pltpu.sync_copy(idx_hbm.at[pl.ds(off, C)], idx_v)
        pltpu.async_copy(tab_hbm.at[idx_v], row_v, sem).wait()
        pltpu.sync_copy(row_v, out_hbm.at[pl.ds(off, C)])
        return 0
    lax.fori_loop(0, NCHUNK, _body, 0)


@functools.partial(
    pl.kernel,
    out_type=jax.ShapeDtypeStruct((NC, N, D), jnp.float32),
    mesh=_MESH,
    compiler_params=pltpu.CompilerParams(needs_layout_passes=False),
    scratch_types=[
        pltpu.VMEM((C,), jnp.int32),
        pltpu.VMEM((C, D), jnp.float32),
        pltpu.VMEM_SHARED((N, D), jnp.float32),
    ],
)
def _scatter_sc(m_hbm, dst_hbm, out_hbm, idx_v, row_v, acc_sh):
    cid = lax.axis_index("c")
    sid = lax.axis_index("s")

    def _zrow(i, _):
        row_v[i // 8, pl.ds((i % 8) * 16, 16)] = jnp.zeros((16,), jnp.float32)
        return 0
    lax.fori_loop(0, C * 8, _zrow, 0)
    slab = sid * 640

    @pl.when(sid < 15)
    def _():
        pltpu.sync_copy(row_v.at[pl.ds(0, 320)], acc_sh.at[pl.ds(slab, 320)])
        pltpu.sync_copy(row_v.at[pl.ds(0, 320)], acc_sh.at[pl.ds(slab + 320, 320)])

    @pl.when(sid == 15)
    def _():
        pltpu.sync_copy(row_v.at[pl.ds(0, 200)], acc_sh.at[pl.ds(slab, 200)])
        pltpu.sync_copy(row_v.at[pl.ds(0, 200)], acc_sh.at[pl.ds(slab + 200, 200)])

    plsc.subcore_barrier()

    per_tile = E_PAD // NC // NS
    base = cid * (E_PAD // NC) + sid * per_tile
    def _body(i, _):
        off = pl.multiple_of(base + i * C, C)
        pltpu.sync_copy(dst_hbm.at[pl.ds(off, C)], idx_v)
        pltpu.sync_copy(m_hbm.at[pl.ds(off, C)], row_v)
        pltpu.sync_copy(row_v, acc_sh.at[idx_v], add=True)
        return 0
    lax.fori_loop(0, per_tile // C, _body, 0)

    plsc.subcore_barrier()

    @pl.when(sid < 15)
    def _():
        pltpu.sync_copy(acc_sh.at[pl.ds(slab, 640)], out_hbm.at[cid, pl.ds(slab, 640)])

    @pl.when(sid == 15)
    def _():
        pltpu.sync_copy(acc_sh.at[pl.ds(slab, 400)], out_hbm.at[cid, pl.ds(slab, 400)])


C2 = 192
K2 = 27
E_PAD2 = NW * C2 * K2

@functools.partial(
    pl.kernel,
    out_type=(jax.ShapeDtypeStruct((E_PAD2,), jnp.float32),
              jax.ShapeDtypeStruct((E_PAD2,), jnp.float32)),
    mesh=_MESH,
    compiler_params=pltpu.CompilerParams(needs_layout_passes=False),
    scratch_types=[
        pltpu.VMEM((C2,), jnp.int32),
        pltpu.VMEM((C2, D), jnp.float32),
        pltpu.VMEM((C2, D), jnp.float32),
        pltpu.VMEM((C2, D), jnp.float32),
        pltpu.VMEM((C2, D), jnp.float32),
        pltpu.VMEM((C2 * 16,), jnp.float32),
        pltpu.VMEM((C2 * 16,), jnp.float32),
        pltpu.VMEM((C2,), jnp.float32),
        pltpu.VMEM((C2,), jnp.float32),
        pltpu.SemaphoreType.DMA,
    ],
)
def _decode_sc(x_hbm, remb_hbm, head_hbm, tail_hbm, ntail_hbm, rel_hbm,
               pos_hbm, neg_hbm,
               idx_v, xh_v, re_v, xt_v, xn_v, ap_v, an_v, pb_v, nb_v, sem):
    base = _wid() * C2 * K2
    lanes = lax.iota(jnp.int32, 16)
    def _body(i, _):
        off = pl.multiple_of(base + i * C2, 8)
        pltpu.sync_copy(head_hbm.at[pl.ds(off, C2)], idx_v)
        pltpu.async_copy(x_hbm.at[idx_v], xh_v, sem).wait()
        pltpu.sync_copy(rel_hbm.at[pl.ds(off, C2)], idx_v)
        pltpu.async_copy(remb_hbm.at[idx_v], re_v, sem).wait()
        pltpu.sync_copy(tail_hbm.at[pl.ds(off, C2)], idx_v)
        pltpu.async_copy(x_hbm.at[idx_v], xt_v, sem).wait()
        pltpu.sync_copy(ntail_hbm.at[pl.ds(off, C2)], idx_v)
        pltpu.async_copy(x_hbm.at[idx_v], xn_v, sem).wait()

        def _row(r, _):
            accp = jnp.zeros((16,), jnp.float32)
            accn = jnp.zeros((16,), jnp.float32)
            for k in range(D // 16):
                sl = pl.ds(k * 16, 16)
                h = xh_v[r, sl] * re_v[r, sl]
                accp = accp + h * xt_v[r, sl]
                accn = accn + h * xn_v[r, sl]
            ap_v[pl.ds(r * 16, 16)] = accp
            an_v[pl.ds(r * 16, 16)] = accn
            return 0
        lax.fori_loop(0, C2, _row, 0)

        def _red(g, _):
            idx0 = (g * 16 + lanes) * 16
            ps = jnp.zeros((16,), jnp.float32)
            ns = jnp.zeros((16,), jnp.float32)
            for j in range(16):
                ps = ps + plsc.load_gather(ap_v, [idx0 + j])
                ns = ns + plsc.load_gather(an_v, [idx0 + j])
            pb_v[pl.ds(g * 16, 16)] = ps
            nb_v[pl.ds(g * 16, 16)] = ns
            return 0
        lax.fori_loop(0, C2 // 16, _red, 0)

        pltpu.sync_copy(pb_v, pos_hbm.at[pl.ds(off, C2)])
        pltpu.sync_copy(nb_v, neg_hbm.at[pl.ds(off, C2)])
        return 0
    lax.fori_loop(0, K2, _body, 0)


CM = 1024


def _basis_body(x_ref, n_ref, rel_ref, comb_ref, bas_ref, o_ref):
    xn = x_ref[...] * n_ref[...]
    rel = rel_ref[...]
    onehot = (lax.broadcasted_iota(jnp.int32, (CM, 128), 1) == rel
              ).astype(jnp.float32)
    coef = jnp.dot(onehot, comb_ref[...],
                   preferred_element_type=jnp.float32)
    xb = jnp.concatenate([xn * coef[:, b][:, None] for b in range(B)], axis=1)
    o_ref[...] = jnp.dot(xb, bas_ref[...], preferred_element_type=jnp.float32)


def _basis_tc(xsrc, norm2d, rel2d, comb_p, bases_r):
    return pl.pallas_call(
        _basis_body,
        grid=(E_PAD // CM,),
        in_specs=[
            pl.BlockSpec((CM, D), lambda i: (i, 0)),
            pl.BlockSpec((CM, 1), lambda i: (i, 0)),
            pl.BlockSpec((CM, 1), lambda i: (i, 0)),
            pl.BlockSpec((128, B), lambda i: (0, 0)),
            pl.BlockSpec((B * D, H), lambda i: (0, 0)),
        ],
        out_specs=pl.BlockSpec((CM, H), lambda i: (i, 0)),
        out_shape=jax.ShapeDtypeStruct((E_PAD, H), jnp.float32),
    )(xsrc, norm2d, rel2d, comb_p, bases_r)


def _merge_body(relu, p_ref, b_ref, o_ref):
    s = p_ref[0] + p_ref[1] + b_ref[...]
    o_ref[...] = jnp.maximum(s, 0.0) if relu else s


def _merge_tc(partial, bias2d, relu):
    return pl.pallas_call(
        functools.partial(_merge_body, relu),
        grid=(10,),
        in_specs=[
            pl.BlockSpec((NC, 1000, D), lambda i: (0, i, 0)),
            pl.BlockSpec((1, D), lambda i: (0, 0)),
        ],
        out_specs=pl.BlockSpec((1000, D), lambda i: (i, 0)),
        out_shape=jax.ShapeDtypeStruct((N, D), jnp.float32),
    )(partial, bias2d)


def _x0_body(e_ref, b_ref, o_ref):
    o_ref[...] = jnp.maximum(e_ref[...] + b_ref[...], 0.0)


def _x0_tc(emb, ebias):
    return pl.pallas_call(
        _x0_body,
        grid=(10,),
        in_specs=[
            pl.BlockSpec((1000, D), lambda i: (i, 0)),
            pl.BlockSpec((1, D), lambda i: (0, 0)),
        ],
        out_specs=pl.BlockSpec((1000, D), lambda i: (i, 0)),
        out_shape=jax.ShapeDtypeStruct((N, D), jnp.float32),
    )(emb, ebias)


def kernel(entity_embedding, entity_embedding_bias, bases1, comb1, b1,
           bases2, comb2, b2, rel_emb, edge_index, edge_type):
    i32 = jnp.int32
    src = edge_index[0].astype(i32)
    dst = edge_index[1].astype(i32)
    rel = edge_type.astype(i32)
    loop = jnp.arange(N, dtype=i32)

    npad = E_PAD - E_F
    src_f = jnp.concatenate([src, dst, loop, jnp.zeros((npad,), i32)])
    dst_f = jnp.concatenate([dst, src, loop, jnp.zeros((npad,), i32)])
    rel_f = jnp.concatenate([rel, rel + R, jnp.full((N,), 2 * R, i32),
                             jnp.full((npad,), NUM_REL, i32)])
    key_f = dst_f * 128 + rel_f

    comb1p = jnp.zeros((128, B), jnp.float32).at[:NUM_REL].set(comb1)
    comb2p = jnp.zeros((128, B), jnp.float32).at[:NUM_REL].set(comb2)

    cnt = _count_sc(key_f).reshape(ROWS_PAD, 128)

    x = _x0_tc(entity_embedding, entity_embedding_bias)

    rel2d = rel_f.reshape(E_PAD, 1)
    bases1_r = bases1.reshape(B * D, H)
    bases2_r = bases2.reshape(B * H, D)

    xsrc, norm = _gather_norm_sc(x, cnt, src_f, dst_f, rel_f)
    norm2d = norm.reshape(E_PAD, 1)
    m = _basis_tc(xsrc, norm2d, rel2d, comb1p, bases1_r)
    partial = _scatter_sc(m, dst_f)
    x = _merge_tc(partial, b1.reshape(1, H), relu=True)

    xsrc = _gather_sc(x, src_f)
    m = _basis_tc(xsrc, norm2d, rel2d, comb2p, bases2_r)
    partial = _scatter_sc(m, dst_f)
    x = _merge_tc(partial, b2.reshape(1, D), relu=False)

    neg_tail = jax.random.randint(jax.random.key(42), (E,), 0, N).astype(i32)
    dpad = E_PAD2 - E
    zpad = jnp.zeros((dpad,), i32)
    pos, neg = _decode_sc(x, rel_emb,
                          jnp.concatenate([src, zpad]),
                          jnp.concatenate([dst, zpad]),
                          jnp.concatenate([neg_tail, zpad]),
                          jnp.concatenate([rel, zpad]))
    pos, neg = pos[:E], neg[:E]

    loss = (jnp.sum(jax.nn.softplus(-pos)) + jnp.sum(jax.nn.softplus(neg))) / (2.0 * E)
    cnt = jnp.searchsorted(jnp.sort(neg), pos, side="left")
    auc = jnp.sum(cnt.astype(jnp.float32)) / (float(E) * float(E))
    return (pos, loss, auc)


# speedup vs baseline: 5.2027x; 1.2685x over previous
import functools

import jax
import jax.numpy as jnp
from jax import lax
from jax.experimental import pallas as pl
from jax.experimental.pallas import tpu as pltpu
from jax.experimental.pallas import tpu_sc as plsc

N = 10000
R = 50
NUM_REL = 2 * R + 1
D = 128
H = 128
B = 8
E = 160000
E_F = 2 * E + N

NC = 2
NS = 16
NW = NC * NS
C = 384
NCHUNK = 28
E_PAD = NW * C * NCHUNK

ROWS_PAD = 10240
RPT = ROWS_PAD // NW
C1 = 2048

_MESH = plsc.VectorSubcoreMesh(core_axis_name="c", subcore_axis_name="s",
                               num_cores=NC, num_subcores=NS)


def _wid():
    return lax.axis_index("c") * NS + lax.axis_index("s")


@functools.partial(
    pl.kernel,
    out_type=jax.ShapeDtypeStruct((ROWS_PAD * 128,), jnp.float32),
    mesh=_MESH,
    name="sc_count",
    compiler_params=pltpu.CompilerParams(needs_layout_passes=False),
    scratch_types=[
        pltpu.VMEM((C1,), jnp.int32),
        pltpu.VMEM((RPT * 128,), jnp.float32),
    ],
)
def _count_sc(key_hbm, cnt_hbm, key_v, cnt_t):
    wid = _wid()
    key_lo = wid * RPT * 128

    def _zero(i, _):
        cnt_t[pl.ds(i * 16, 16)] = jnp.zeros((16,), jnp.float32)
        return 0
    lax.fori_loop(0, RPT * 8, _zero, 0)

    ones = jnp.ones((16,), jnp.float32)
    def _chunk(ch, _):
        pltpu.sync_copy(key_hbm.at[pl.ds(ch * C1, C1)], key_v)
        def _grp(g, _):
            kv = key_v[pl.ds(g * 16, 16)]
            lkey = kv - key_lo
            msk = (kv >= key_lo) & (kv < key_lo + RPT * 128)
            plsc.addupdate_scatter(cnt_t, [lkey], ones, mask=msk)
            return 0
        lax.fori_loop(0, C1 // 16, _grp, 0)
        return 0
    lax.fori_loop(0, E_PAD // C1, _chunk, 0)

    pltpu.sync_copy(cnt_t, cnt_hbm.at[pl.ds(key_lo, RPT * 128)])


@functools.partial(
    pl.kernel,
    out_type=(jax.ShapeDtypeStruct((E_PAD, D), jnp.float32),
              jax.ShapeDtypeStruct((E_PAD,), jnp.float32)),
    mesh=_MESH,
    name="sc_gather_norm",
    compiler_params=pltpu.CompilerParams(needs_layout_passes=False),
    scratch_types=[
        pltpu.VMEM((C,), jnp.int32),
        pltpu.VMEM((C,), jnp.int32),
        pltpu.VMEM((C,), jnp.int32),
        pltpu.VMEM((C, D), jnp.float32),
        pltpu.VMEM((C, 128), jnp.float32),
        pltpu.VMEM((C,), jnp.float32),
        pltpu.SemaphoreType.DMA,
        pltpu.SemaphoreType.DMA,
    ],
)
def _gather_norm_sc(tab_hbm, cnt_hbm, src_hbm, dst_hbm, rel_hbm,
                    out_hbm, norm_hbm,
                    idx_v, dst_v, rel_v, row_v, crow_v, nrm_v, sem1, sem2):
    base = _wid() * C * NCHUNK
    lanes = lax.iota(jnp.int32, 16)
    def _body(i, _):
        off = pl.multiple_of(base + i * C, C)
        pltpu.sync_copy(src_hbm.at[pl.ds(off, C)], idx_v)
        cp1 = pltpu.async_copy(tab_hbm.at[idx_v], row_v, sem1)
        pltpu.sync_copy(dst_hbm.at[pl.ds(off, C)], dst_v)
        cp2 = pltpu.async_copy(cnt_hbm.at[dst_v], crow_v, sem2)
        pltpu.sync_copy(rel_hbm.at[pl.ds(off, C)], rel_v)
        cp1.wait()
        pltpu.sync_copy(row_v, out_hbm.at[pl.ds(off, C)])
        cp2.wait()
        def _grp(g, _):
            rows = g * 16 + lanes
            lane = rel_v[pl.ds(g * 16, 16)]
            cv = plsc.load_gather(crow_v, [rows, lane])
            nrm_v[pl.ds(g * 16, 16)] = 1.0 / jnp.maximum(cv, 1.0)
            return 0
        lax.fori_loop(0, C // 16, _grp, 0)
        pltpu.sync_copy(nrm_v, norm_hbm.at[pl.ds(off, C)])
        return 0
    lax.fori_loop(0, NCHUNK, _body, 0)


@functools.partial(
    pl.kernel,
    out_type=jax.ShapeDtypeStruct((E_PAD, D), jnp.float32),
    mesh=_MESH,
    name="sc_gather",
    compiler_params=pltpu.CompilerParams(needs_layout_passes=False),
    scratch_types=[
        pltpu.VMEM((C,), jnp.int32),
        pltpu.VMEM((C, D), jnp.float32),
        pltpu.SemaphoreType.DMA,
    ],
)
def _gather_sc(tab_hbm, idx_hbm, out_hbm, idx_v, row_v, sem):
    base = _wid() * C * NCHUNK
    def _body(i, _):
        off = pl.multiple_of(base + i * C, C)
        pltpu.sync_copy(idx_hbm.at[pl.ds(off, C)], idx_v)
        pltpu.async_copy(tab_hbm.at[idx_v], row_v, sem).wait()
        pltpu.sync_copy(row_v, out_hbm.at[pl.ds(off, C)])
        return 0
    lax.fori_loop(0, NCHUNK, _body, 0)


@functools.partial(
    pl.kernel,
    out_type=jax.ShapeDtypeStruct((NC, N, D), jnp.float32),
    mesh=_MESH,
    name="sc_scatter",
    compiler_params=pltpu.CompilerParams(needs_layout_passes=False),
    scratch_types=[
        pltpu.VMEM((C,), jnp.int32),
        pltpu.VMEM((C, D), jnp.float32),
        pltpu.VMEM_SHARED((N, D), jnp.float32),
    ],
)
def _scatter_sc(m_hbm, dst_hbm, out_hbm, idx_v, row_v, acc_sh):
    cid = lax.axis_index("c")
    sid = lax.axis_index("s")

    def _zrow(i, _):
        row_v[i // 8, pl.ds((i % 8) * 16, 16)] = jnp.zeros((16,), jnp.float32)
        return 0
    lax.fori_loop(0, C * 8, _zrow, 0)
    slab = sid * 640

    @pl.when(sid < 15)
    def _():
        pltpu.sync_copy(row_v.at[pl.ds(0, 320)], acc_sh.at[pl.ds(slab, 320)])
        pltpu.sync_copy(row_v.at[pl.ds(0, 320)], acc_sh.at[pl.ds(slab + 320, 320)])

    @pl.when(sid == 15)
    def _():
        pltpu.sync_copy(row_v.at[pl.ds(0, 200)], acc_sh.at[pl.ds(slab, 200)])
        pltpu.sync_copy(row_v.at[pl.ds(0, 200)], acc_sh.at[pl.ds(slab + 200, 200)])

    plsc.subcore_barrier()

    per_tile = E_PAD // NC // NS
    base = cid * (E_PAD // NC) + sid * per_tile
    def _body(i, _):
        off = pl.multiple_of(base + i * C, C)
        pltpu.sync_copy(dst_hbm.at[pl.ds(off, C)], idx_v)
        pltpu.sync_copy(m_hbm.at[pl.ds(off, C)], row_v)
        pltpu.sync_copy(row_v, acc_sh.at[idx_v], add=True)
        return 0
    lax.fori_loop(0, per_tile // C, _body, 0)

    plsc.subcore_barrier()

    @pl.when(sid < 15)
    def _():
        pltpu.sync_copy(acc_sh.at[pl.ds(slab, 640)], out_hbm.at[cid, pl.ds(slab, 640)])

    @pl.when(sid == 15)
    def _():
        pltpu.sync_copy(acc_sh.at[pl.ds(slab, 400)], out_hbm.at[cid, pl.ds(slab, 400)])


C2 = 192
K2 = 27
E_PAD2 = NW * C2 * K2

@functools.partial(
    pl.kernel,
    out_type=(jax.ShapeDtypeStruct((E_PAD2,), jnp.float32),
              jax.ShapeDtypeStruct((E_PAD2,), jnp.float32)),
    mesh=_MESH,
    name="sc_decode",
    compiler_params=pltpu.CompilerParams(needs_layout_passes=False),
    scratch_types=[
        pltpu.VMEM((C2,), jnp.int32),
        pltpu.VMEM((C2, D), jnp.float32),
        pltpu.VMEM((C2, D), jnp.float32),
        pltpu.VMEM((C2, D), jnp.float32),
        pltpu.VMEM((C2, D), jnp.float32),
        pltpu.VMEM((C2 * 16,), jnp.float32),
        pltpu.VMEM((C2 * 16,), jnp.float32),
        pltpu.VMEM((C2,), jnp.float32),
        pltpu.VMEM((C2,), jnp.float32),
        pltpu.SemaphoreType.DMA,
    ],
)
def _decode_sc(x_hbm, remb_hbm, head_hbm, tail_hbm, ntail_hbm, rel_hbm,
               pos_hbm, neg_hbm,
               idx_v, xh_v, re_v, xt_v, xn_v, ap_v, an_v, pb_v, nb_v, sem):
    base = _wid() * C2 * K2
    lanes = lax.iota(jnp.int32, 16)
    def _body(i, _):
        off = pl.multiple_of(base + i * C2, 8)
        pltpu.sync_copy(head_hbm.at[pl.ds(off, C2)], idx_v)
        pltpu.async_copy(x_hbm.at[idx_v], xh_v, sem).wait()
        pltpu.sync_copy(rel_hbm.at[pl.ds(off, C2)], idx_v)
        pltpu.async_copy(remb_hbm.at[idx_v], re_v, sem).wait()
        pltpu.sync_copy(tail_hbm.at[pl.ds(off, C2)], idx_v)
        pltpu.async_copy(x_hbm.at[idx_v], xt_v, sem).wait()
        pltpu.sync_copy(ntail_hbm.at[pl.ds(off, C2)], idx_v)
        pltpu.async_copy(x_hbm.at[idx_v], xn_v, sem).wait()

        def _row(r, _):
            accp = jnp.zeros((16,), jnp.float32)
            accn = jnp.zeros((16,), jnp.float32)
            for k in range(D // 16):
                sl = pl.ds(k * 16, 16)
                h = xh_v[r, sl] * re_v[r, sl]
                accp = accp + h * xt_v[r, sl]
                accn = accn + h * xn_v[r, sl]
            ap_v[pl.ds(r * 16, 16)] = accp
            an_v[pl.ds(r * 16, 16)] = accn
            return 0
        lax.fori_loop(0, C2, _row, 0)

        def _red(g, _):
            idx0 = (g * 16 + lanes) * 16
            ps = jnp.zeros((16,), jnp.float32)
            ns = jnp.zeros((16,), jnp.float32)
            for j in range(16):
                ps = ps + plsc.load_gather(ap_v, [idx0 + j])
                ns = ns + plsc.load_gather(an_v, [idx0 + j])
            pb_v[pl.ds(g * 16, 16)] = ps
            nb_v[pl.ds(g * 16, 16)] = ns
            return 0
        lax.fori_loop(0, C2 // 16, _red, 0)

        pltpu.sync_copy(pb_v, pos_hbm.at[pl.ds(off, C2)])
        pltpu.sync_copy(nb_v, neg_hbm.at[pl.ds(off, C2)])
        return 0
    lax.fori_loop(0, K2, _body, 0)


CM = 1024


def _basis_body(x_ref, n_ref, rel_ref, comb_ref, bas_ref, o_ref):
    xn = x_ref[...] * n_ref[...]
    rel = rel_ref[...]
    onehot = (lax.broadcasted_iota(jnp.int32, (CM, 128), 1) == rel
              ).astype(jnp.float32)
    coef = jnp.dot(onehot, comb_ref[...],
                   preferred_element_type=jnp.float32)
    xb = jnp.concatenate([xn * coef[:, b][:, None] for b in range(B)], axis=1)
    o_ref[...] = jnp.dot(xb, bas_ref[...], preferred_element_type=jnp.float32)


def _basis_tc(xsrc, norm2d, rel2d, comb_p, bases_r):
    return pl.pallas_call(
        _basis_body,
        grid=(E_PAD // CM,),
        in_specs=[
            pl.BlockSpec((CM, D), lambda i: (i, 0)),
            pl.BlockSpec((CM, 1), lambda i: (i, 0)),
            pl.BlockSpec((CM, 1), lambda i: (i, 0)),
            pl.BlockSpec((128, B), lambda i: (0, 0)),
            pl.BlockSpec((B * D, H), lambda i: (0, 0)),
        ],
        out_specs=pl.BlockSpec((CM, H), lambda i: (i, 0)),
        out_shape=jax.ShapeDtypeStruct((E_PAD, H), jnp.float32),
    )(xsrc, norm2d, rel2d, comb_p, bases_r)


def _merge_body(relu, p_ref, b_ref, o_ref):
    s = p_ref[0] + p_ref[1] + b_ref[...]
    o_ref[...] = jnp.maximum(s, 0.0) if relu else s


def _merge_tc(partial, bias2d, relu):
    return pl.pallas_call(
        functools.partial(_merge_body, relu),
        grid=(10,),
        in_specs=[
            pl.BlockSpec((NC, 1000, D), lambda i: (0, i, 0)),
            pl.BlockSpec((1, D), lambda i: (0, 0)),
        ],
        out_specs=pl.BlockSpec((1000, D), lambda i: (i, 0)),
        out_shape=jax.ShapeDtypeStruct((N, D), jnp.float32),
    )(partial, bias2d)


def _x0_body(e_ref, b_ref, o_ref):
    o_ref[...] = jnp.maximum(e_ref[...] + b_ref[...], 0.0)


def _x0_tc(emb, ebias):
    return pl.pallas_call(
        _x0_body,
        grid=(10,),
        in_specs=[
            pl.BlockSpec((1000, D), lambda i: (i, 0)),
            pl.BlockSpec((1, D), lambda i: (0, 0)),
        ],
        out_specs=pl.BlockSpec((1000, D), lambda i: (i, 0)),
        out_shape=jax.ShapeDtypeStruct((N, D), jnp.float32),
    )(emb, ebias)


def kernel(entity_embedding, entity_embedding_bias, bases1, comb1, b1,
           bases2, comb2, b2, rel_emb, edge_index, edge_type):
    i32 = jnp.int32
    src = edge_index[0].astype(i32)
    dst = edge_index[1].astype(i32)
    rel = edge_type.astype(i32)
    loop = jnp.arange(N, dtype=i32)

    npad = E_PAD - E_F
    src_f = jnp.concatenate([src, dst, loop, jnp.zeros((npad,), i32)])
    dst_f = jnp.concatenate([dst, src, loop, jnp.zeros((npad,), i32)])
    rel_f = jnp.concatenate([rel, rel + R, jnp.full((N,), 2 * R, i32),
                             jnp.full((npad,), NUM_REL, i32)])
    key_f = dst_f * 128 + rel_f

    comb1p = jnp.zeros((128, B), jnp.float32).at[:NUM_REL].set(comb1)
    comb2p = jnp.zeros((128, B), jnp.float32).at[:NUM_REL].set(comb2)

    cnt = _count_sc(key_f).reshape(ROWS_PAD, 128)

    x = _x0_tc(entity_embedding, entity_embedding_bias)

    rel2d = rel_f.reshape(E_PAD, 1)
    bases1_r = bases1.reshape(B * D, H)
    bases2_r = bases2.reshape(B * H, D)

    xsrc, norm = _gather_norm_sc(x, cnt, src_f, dst_f, rel_f)
    norm2d = norm.reshape(E_PAD, 1)
    m = _basis_tc(xsrc, norm2d, rel2d, comb1p, bases1_r)
    partial = _scatter_sc(m, dst_f)
    x = _merge_tc(partial, b1.reshape(1, H), relu=True)

    xsrc = _gather_sc(x, src_f)
    m = _basis_tc(xsrc, norm2d, rel2d, comb2p, bases2_r)
    partial = _scatter_sc(m, dst_f)
    x = _merge_tc(partial, b2.reshape(1, D), relu=False)

    neg_tail = jax.random.randint(jax.random.key(42), (E,), 0, N).astype(i32)
    dpad = E_PAD2 - E
    zpad = jnp.zeros((dpad,), i32)
    pos, neg = _decode_sc(x, rel_emb,
                          jnp.concatenate([src, zpad]),
                          jnp.concatenate([dst, zpad]),
                          jnp.concatenate([neg_tail, zpad]),
                          jnp.concatenate([rel, zpad]))
    pos, neg = pos[:E], neg[:E]

    loss = (jnp.sum(jax.nn.softplus(-pos)) + jnp.sum(jax.nn.softplus(neg))) / (2.0 * E)
    flags = jnp.concatenate([jnp.zeros((E,), jnp.int32), jnp.ones((E,), jnp.int32)])
    _, sflags = lax.sort((jnp.concatenate([pos, neg]), flags), num_keys=2)
    nb = jnp.cumsum(sflags) - sflags
    per_pos = jnp.where(sflags == 0, nb, 0)
    hi = jnp.sum(per_pos >> 12)
    lo = jnp.sum(per_pos & 0xFFF)
    auc = (hi.astype(jnp.float32) * 4096.0 + lo.astype(jnp.float32)) / (float(E) * float(E))
    return (pos, loss, auc)


# speedup vs baseline: 5.6035x; 1.0770x over previous
import functools

import jax
import jax.numpy as jnp
from jax import lax
from jax.experimental import pallas as pl
from jax.experimental.pallas import tpu as pltpu
from jax.experimental.pallas import tpu_sc as plsc

N = 10000
R = 50
NUM_REL = 2 * R + 1
D = 128
H = 128
B = 8
E = 160000
E_F = 2 * E + N

NC = 2
NS = 16
NW = NC * NS
C = 384
NCHUNK = 28
E_PAD = NW * C * NCHUNK

ROWS_PAD = 10240
RPT = ROWS_PAD // NW
C1 = 2048

_MESH = plsc.VectorSubcoreMesh(core_axis_name="c", subcore_axis_name="s",
                               num_cores=NC, num_subcores=NS)


def _wid():
    return lax.axis_index("c") * NS + lax.axis_index("s")


@functools.partial(
    pl.kernel,
    out_type=jax.ShapeDtypeStruct((ROWS_PAD * 128,), jnp.float32),
    mesh=_MESH,
    name="sc_count",
    compiler_params=pltpu.CompilerParams(needs_layout_passes=False),
    scratch_types=[
        pltpu.VMEM((C1,), jnp.int32),
        pltpu.VMEM((RPT * 128,), jnp.float32),
    ],
)
def _count_sc(key_hbm, cnt_hbm, key_v, cnt_t):
    wid = _wid()
    key_lo = wid * RPT * 128

    def _zero(i, _):
        cnt_t[pl.ds(i * 16, 16)] = jnp.zeros((16,), jnp.float32)
        return 0
    lax.fori_loop(0, RPT * 8, _zero, 0)

    ones = jnp.ones((16,), jnp.float32)
    def _chunk(ch, _):
        pltpu.sync_copy(key_hbm.at[pl.ds(ch * C1, C1)], key_v)
        def _grp(g, _):
            kv = key_v[pl.ds(g * 16, 16)]
            lkey = kv - key_lo
            msk = (kv >= key_lo) & (kv < key_lo + RPT * 128)
            plsc.addupdate_scatter(cnt_t, [lkey], ones, mask=msk)
            return 0
        lax.fori_loop(0, C1 // 16, _grp, 0)
        return 0
    lax.fori_loop(0, E_PAD // C1, _chunk, 0)

    pltpu.sync_copy(cnt_t, cnt_hbm.at[pl.ds(key_lo, RPT * 128)])


CN = 192
NCHUNK_N = 56

@functools.partial(
    pl.kernel,
    out_type=(jax.ShapeDtypeStruct((E_PAD, D), jnp.float32),
              jax.ShapeDtypeStruct((E_PAD,), jnp.float32)),
    mesh=_MESH,
    name="sc_gather_norm",
    compiler_params=pltpu.CompilerParams(needs_layout_passes=False),
    scratch_types=[
        pltpu.VMEM((CN,), jnp.int32), pltpu.VMEM((CN,), jnp.int32),
        pltpu.VMEM((CN,), jnp.int32), pltpu.VMEM((CN,), jnp.int32),
        pltpu.VMEM((CN,), jnp.int32),
        pltpu.VMEM((CN, D), jnp.float32), pltpu.VMEM((CN, D), jnp.float32),
        pltpu.VMEM((CN, 128), jnp.float32), pltpu.VMEM((CN, 128), jnp.float32),
        pltpu.VMEM((CN,), jnp.float32),
        pltpu.SemaphoreType.DMA, pltpu.SemaphoreType.DMA,
        pltpu.SemaphoreType.DMA, pltpu.SemaphoreType.DMA,
    ],
)
def _gather_norm_sc(tab_hbm, cnt_hbm, src_hbm, dst_hbm, rel_hbm,
                    out_hbm, norm_hbm,
                    is0, is1, id0, id1, rel_v, xr0, xr1, cr0, cr1, nrm_v,
                    sx0, sx1, sc0, sc1):
    base = _wid() * CN * NCHUNK_N
    lanes = lax.iota(jnp.int32, 16)

    def _start(off, is_v, id_v, xr_v, cr_v, semx, semc):
        pltpu.sync_copy(src_hbm.at[pl.ds(off, CN)], is_v)
        pltpu.async_copy(tab_hbm.at[is_v], xr_v, semx)
        pltpu.sync_copy(dst_hbm.at[pl.ds(off, CN)], id_v)
        pltpu.async_copy(cnt_hbm.at[id_v], cr_v, semc)

    def _drain(off, is_v, id_v, xr_v, cr_v, semx, semc):
        pltpu.make_async_copy(tab_hbm.at[is_v], xr_v, semx).wait()
        pltpu.sync_copy(xr_v, out_hbm.at[pl.ds(off, CN)])
        pltpu.make_async_copy(cnt_hbm.at[id_v], cr_v, semc).wait()
        pltpu.sync_copy(rel_hbm.at[pl.ds(off, CN)], rel_v)
        def _grp(g, _):
            rows = g * 16 + lanes
            lane = rel_v[pl.ds(g * 16, 16)]
            cv = plsc.load_gather(cr_v, [rows, lane])
            nrm_v[pl.ds(g * 16, 16)] = 1.0 / jnp.maximum(cv, 1.0)
            return 0
        lax.fori_loop(0, CN // 16, _grp, 0)
        pltpu.sync_copy(nrm_v, norm_hbm.at[pl.ds(off, CN)])

    _start(pl.multiple_of(base, CN), is0, id0, xr0, cr0, sx0, sc0)
    def _body(jj, _):
        offa = pl.multiple_of(base + (2 * jj) * CN, CN)
        offb = pl.multiple_of(base + (2 * jj + 1) * CN, CN)
        _start(offb, is1, id1, xr1, cr1, sx1, sc1)
        _drain(offa, is0, id0, xr0, cr0, sx0, sc0)
        @pl.when(2 * jj + 2 < NCHUNK_N)
        def _():
            offc = pl.multiple_of(base + (2 * jj + 2) * CN, CN)
            _start(offc, is0, id0, xr0, cr0, sx0, sc0)
        _drain(offb, is1, id1, xr1, cr1, sx1, sc1)
        return 0
    lax.fori_loop(0, NCHUNK_N // 2, _body, 0)


@functools.partial(
    pl.kernel,
    out_type=jax.ShapeDtypeStruct((E_PAD, D), jnp.float32),
    mesh=_MESH,
    name="sc_gather",
    compiler_params=pltpu.CompilerParams(needs_layout_passes=False),
    scratch_types=[
        pltpu.VMEM((C,), jnp.int32), pltpu.VMEM((C,), jnp.int32),
        pltpu.VMEM((C, D), jnp.float32), pltpu.VMEM((C, D), jnp.float32),
        pltpu.SemaphoreType.DMA, pltpu.SemaphoreType.DMA,
    ],
)
def _gather_sc(tab_hbm, idx_hbm, out_hbm, i0, i1, r0, r1, s0, s1):
    base = _wid() * C * NCHUNK

    def _start(off, i_v, r_v, sem):
        pltpu.sync_copy(idx_hbm.at[pl.ds(off, C)], i_v)
        pltpu.async_copy(tab_hbm.at[i_v], r_v, sem)

    def _drain(off, i_v, r_v, sem):
        pltpu.make_async_copy(tab_hbm.at[i_v], r_v, sem).wait()
        pltpu.sync_copy(r_v, out_hbm.at[pl.ds(off, C)])

    _start(pl.multiple_of(base, C), i0, r0, s0)
    def _body(jj, _):
        offa = pl.multiple_of(base + (2 * jj) * C, C)
        offb = pl.multiple_of(base + (2 * jj + 1) * C, C)
        _start(offb, i1, r1, s1)
        _drain(offa, i0, r0, s0)
        @pl.when(2 * jj + 2 < NCHUNK)
        def _():
            offc = pl.multiple_of(base + (2 * jj + 2) * C, C)
            _start(offc, i0, r0, s0)
        _drain(offb, i1, r1, s1)
        return 0
    lax.fori_loop(0, NCHUNK // 2, _body, 0)


@functools.partial(
    pl.kernel,
    out_type=jax.ShapeDtypeStruct((NC, N, D), jnp.float32),
    mesh=_MESH,
    name="sc_scatter",
    compiler_params=pltpu.CompilerParams(needs_layout_passes=False),
    scratch_types=[
        pltpu.VMEM((C,), jnp.int32),
        pltpu.VMEM((C, D), jnp.float32),
        pltpu.VMEM_SHARED((N, D), jnp.float32),
    ],
)
def _scatter_sc(m_hbm, dst_hbm, out_hbm, idx_v, row_v, acc_sh):
    cid = lax.axis_index("c")
    sid = lax.axis_index("s")

    def _zrow(i, _):
        row_v[i // 8, pl.ds((i % 8) * 16, 16)] = jnp.zeros((16,), jnp.float32)
        return 0
    lax.fori_loop(0, C * 8, _zrow, 0)
    slab = sid * 640

    @pl.when(sid < 15)
    def _():
        pltpu.sync_copy(row_v.at[pl.ds(0, 320)], acc_sh.at[pl.ds(slab, 320)])
        pltpu.sync_copy(row_v.at[pl.ds(0, 320)], acc_sh.at[pl.ds(slab + 320, 320)])

    @pl.when(sid == 15)
    def _():
        pltpu.sync_copy(row_v.at[pl.ds(0, 200)], acc_sh.at[pl.ds(slab, 200)])
        pltpu.sync_copy(row_v.at[pl.ds(0, 200)], acc_sh.at[pl.ds(slab + 200, 200)])

    plsc.subcore_barrier()

    per_tile = E_PAD // NC // NS
    base = cid * (E_PAD // NC) + sid * per_tile
    def _body(i, _):
        off = pl.multiple_of(base + i * C, C)
        pltpu.sync_copy(dst_hbm.at[pl.ds(off, C)], idx_v)
        pltpu.sync_copy(m_hbm.at[pl.ds(off, C)], row_v)
        pltpu.sync_copy(row_v, acc_sh.at[idx_v], add=True)
        return 0
    lax.fori_loop(0, per_tile // C, _body, 0)

    plsc.subcore_barrier()

    @pl.when(sid < 15)
    def _():
        pltpu.sync_copy(acc_sh.at[pl.ds(slab, 640)], out_hbm.at[cid, pl.ds(slab, 640)])

    @pl.when(sid == 15)
    def _():
        pltpu.sync_copy(acc_sh.at[pl.ds(slab, 400)], out_hbm.at[cid, pl.ds(slab, 400)])


C2 = 192
K2 = 27
E_PAD2 = NW * C2 * K2

@functools.partial(
    pl.kernel,
    out_type=(jax.ShapeDtypeStruct((E_PAD2,), jnp.float32),
              jax.ShapeDtypeStruct((E_PAD2,), jnp.float32)),
    mesh=_MESH,
    name="sc_decode",
    compiler_params=pltpu.CompilerParams(needs_layout_passes=False),
    scratch_types=[
        pltpu.VMEM((C2,), jnp.int32),
        pltpu.VMEM((C2,), jnp.int32),
        pltpu.VMEM((C2,), jnp.int32),
        pltpu.VMEM((C2,), jnp.int32),
        pltpu.VMEM((C2, D), jnp.float32),
        pltpu.VMEM((C2, D), jnp.float32),
        pltpu.VMEM((C2, D), jnp.float32),
        pltpu.VMEM((C2, D), jnp.float32),
        pltpu.VMEM((C2 * 16,), jnp.float32),
        pltpu.VMEM((C2 * 16,), jnp.float32),
        pltpu.VMEM((C2,), jnp.float32),
        pltpu.VMEM((C2,), jnp.float32),
        pltpu.SemaphoreType.DMA, pltpu.SemaphoreType.DMA,
        pltpu.SemaphoreType.DMA, pltpu.SemaphoreType.DMA,
    ],
)
def _decode_sc(x_hbm, remb_hbm, head_hbm, tail_hbm, ntail_hbm, rel_hbm,
               pos_hbm, neg_hbm,
               ih_v, ir_v, it_v, in_v, xh_v, re_v, xt_v, xn_v,
               ap_v, an_v, pb_v, nb_v, s1, s2, s3, s4):
    base = _wid() * C2 * K2
    lanes = lax.iota(jnp.int32, 16)
    def _body(i, _):
        off = pl.multiple_of(base + i * C2, 8)
        pltpu.sync_copy(head_hbm.at[pl.ds(off, C2)], ih_v)
        pltpu.async_copy(x_hbm.at[ih_v], xh_v, s1)
        pltpu.sync_copy(rel_hbm.at[pl.ds(off, C2)], ir_v)
        pltpu.async_copy(remb_hbm.at[ir_v], re_v, s2)
        pltpu.sync_copy(tail_hbm.at[pl.ds(off, C2)], it_v)
        pltpu.async_copy(x_hbm.at[it_v], xt_v, s3)
        pltpu.sync_copy(ntail_hbm.at[pl.ds(off, C2)], in_v)
        pltpu.async_copy(x_hbm.at[in_v], xn_v, s4)
        pltpu.make_async_copy(x_hbm.at[ih_v], xh_v, s1).wait()
        pltpu.make_async_copy(remb_hbm.at[ir_v], re_v, s2).wait()
        pltpu.make_async_copy(x_hbm.at[it_v], xt_v, s3).wait()
        pltpu.make_async_copy(x_hbm.at[in_v], xn_v, s4).wait()

        def _row(r, _):
            accp = jnp.zeros((16,), jnp.float32)
            accn = jnp.zeros((16,), jnp.float32)
            for k in range(D // 16):
                sl = pl.ds(k * 16, 16)
                h = xh_v[r, sl] * re_v[r, sl]
                accp = accp + h * xt_v[r, sl]
                accn = accn + h * xn_v[r, sl]
            ap_v[pl.ds(r * 16, 16)] = accp
            an_v[pl.ds(r * 16, 16)] = accn
            return 0
        lax.fori_loop(0, C2, _row, 0, unroll=4)

        def _red(g, _):
            idx0 = (g * 16 + lanes) * 16
            ps = jnp.zeros((16,), jnp.float32)
            ns = jnp.zeros((16,), jnp.float32)
            for j in range(16):
                ps = ps + plsc.load_gather(ap_v, [idx0 + j])
                ns = ns + plsc.load_gather(an_v, [idx0 + j])
            pb_v[pl.ds(g * 16, 16)] = ps
            nb_v[pl.ds(g * 16, 16)] = ns
            return 0
        lax.fori_loop(0, C2 // 16, _red, 0)

        pltpu.sync_copy(pb_v, pos_hbm.at[pl.ds(off, C2)])
        pltpu.sync_copy(nb_v, neg_hbm.at[pl.ds(off, C2)])
        return 0
    lax.fori_loop(0, K2, _body, 0)


CM = 1024


def _basis_body(x_ref, n_ref, rel_ref, comb_ref, bas_ref, o_ref):
    xn = x_ref[...] * n_ref[...]
    rel = rel_ref[...]
    onehot = (lax.broadcasted_iota(jnp.int32, (CM, 128), 1) == rel
              ).astype(jnp.float32)
    coef = jnp.dot(onehot, comb_ref[...],
                   preferred_element_type=jnp.float32)
    xb = jnp.concatenate([xn * coef[:, b][:, None] for b in range(B)], axis=1)
    o_ref[...] = jnp.dot(xb, bas_ref[...], preferred_element_type=jnp.float32)


def _basis_tc(xsrc, norm2d, rel2d, comb_p, bases_r):
    return pl.pallas_call(
        _basis_body,
        grid=(E_PAD // CM,),
        in_specs=[
            pl.BlockSpec((CM, D), lambda i: (i, 0)),
            pl.BlockSpec((CM, 1), lambda i: (i, 0)),
            pl.BlockSpec((CM, 1), lambda i: (i, 0)),
            pl.BlockSpec((128, B), lambda i: (0, 0)),
            pl.BlockSpec((B * D, H), lambda i: (0, 0)),
        ],
        out_specs=pl.BlockSpec((CM, H), lambda i: (i, 0)),
        out_shape=jax.ShapeDtypeStruct((E_PAD, H), jnp.float32),
    )(xsrc, norm2d, rel2d, comb_p, bases_r)


def _merge_body(relu, p_ref, b_ref, o_ref):
    s = p_ref[0] + p_ref[1] + b_ref[...]
    o_ref[...] = jnp.maximum(s, 0.0) if relu else s


def _merge_tc(partial, bias2d, relu):
    return pl.pallas_call(
        functools.partial(_merge_body, relu),
        grid=(10,),
        in_specs=[
            pl.BlockSpec((NC, 1000, D), lambda i: (0, i, 0)),
            pl.BlockSpec((1, D), lambda i: (0, 0)),
        ],
        out_specs=pl.BlockSpec((1000, D), lambda i: (i, 0)),
        out_shape=jax.ShapeDtypeStruct((N, D), jnp.float32),
    )(partial, bias2d)


def _x0_body(e_ref, b_ref, o_ref):
    o_ref[...] = jnp.maximum(e_ref[...] + b_ref[...], 0.0)


def _x0_tc(emb, ebias):
    return pl.pallas_call(
        _x0_body,
        grid=(10,),
        in_specs=[
            pl.BlockSpec((1000, D), lambda i: (i, 0)),
            pl.BlockSpec((1, D), lambda i: (0, 0)),
        ],
        out_specs=pl.BlockSpec((1000, D), lambda i: (i, 0)),
        out_shape=jax.ShapeDtypeStruct((N, D), jnp.float32),
    )(emb, ebias)


def kernel(entity_embedding, entity_embedding_bias, bases1, comb1, b1,
           bases2, comb2, b2, rel_emb, edge_index, edge_type):
    i32 = jnp.int32
    src = edge_index[0].astype(i32)
    dst = edge_index[1].astype(i32)
    rel = edge_type.astype(i32)
    loop = jnp.arange(N, dtype=i32)

    npad = E_PAD - E_F
    src_f = jnp.concatenate([src, dst, loop, jnp.zeros((npad,), i32)])
    dst_f = jnp.concatenate([dst, src, loop, jnp.zeros((npad,), i32)])
    rel_f = jnp.concatenate([rel, rel + R, jnp.full((N,), 2 * R, i32),
                             jnp.full((npad,), NUM_REL, i32)])
    key_f = dst_f * 128 + rel_f

    comb1p = jnp.zeros((128, B), jnp.float32).at[:NUM_REL].set(comb1)
    comb2p = jnp.zeros((128, B), jnp.float32).at[:NUM_REL].set(comb2)

    cnt = _count_sc(key_f).reshape(ROWS_PAD, 128)

    x = _x0_tc(entity_embedding, entity_embedding_bias)

    rel2d = rel_f.reshape(E_PAD, 1)
    bases1_r = bases1.reshape(B * D, H)
    bases2_r = bases2.reshape(B * H, D)

    xsrc, norm = _gather_norm_sc(x, cnt, src_f, dst_f, rel_f)
    norm2d = norm.reshape(E_PAD, 1)
    m = _basis_tc(xsrc, norm2d, rel2d, comb1p, bases1_r)
    partial = _scatter_sc(m, dst_f)
    x = _merge_tc(partial, b1.reshape(1, H), relu=True)

    xsrc = _gather_sc(x, src_f)
    m = _basis_tc(xsrc, norm2d, rel2d, comb2p, bases2_r)
    partial = _scatter_sc(m, dst_f)
    x = _merge_tc(partial, b2.reshape(1, D), relu=False)

    neg_tail = jax.random.randint(jax.random.key(42), (E,), 0, N).astype(i32)
    dpad = E_PAD2 - E
    zpad = jnp.zeros((dpad,), i32)
    pos, neg = _decode_sc(x, rel_emb,
                          jnp.concatenate([src, zpad]),
                          jnp.concatenate([dst, zpad]),
                          jnp.concatenate([neg_tail, zpad]),
                          jnp.concatenate([rel, zpad]))
    pos, neg = pos[:E], neg[:E]

    loss = (jnp.sum(jax.nn.softplus(-pos)) + jnp.sum(jax.nn.softplus(neg))) / (2.0 * E)
    flags = jnp.concatenate([jnp.zeros((E,), jnp.int32), jnp.ones((E,), jnp.int32)])
    _, sflags = lax.sort((jnp.concatenate([pos, neg]), flags), num_keys=2)
    nb = jnp.cumsum(sflags) - sflags
    per_pos = jnp.where(sflags == 0, nb, 0)
    hi = jnp.sum(per_pos >> 12)
    lo = jnp.sum(per_pos & 0xFFF)
    auc = (hi.astype(jnp.float32) * 4096.0 + lo.astype(jnp.float32)) / (float(E) * float(E))
    return (pos, loss, auc)


# speedup vs baseline: 6.4694x; 1.1545x over previous
import functools

import jax
import jax.numpy as jnp
from jax import lax
from jax.experimental import pallas as pl
from jax.experimental.pallas import tpu as pltpu
from jax.experimental.pallas import tpu_sc as plsc

N = 10000
R = 50
NUM_REL = 2 * R + 1
D = 128
H = 128
B = 8
E = 160000
E_F = 2 * E + N

NC = 2
NS = 16
NW = NC * NS
C = 384
NCHUNK = 28
E_PAD = NW * C * NCHUNK

ROWS_PAD = 10240
RPT = ROWS_PAD // NW
C1 = 2048

_MESH = plsc.VectorSubcoreMesh(core_axis_name="c", subcore_axis_name="s",
                               num_cores=NC, num_subcores=NS)


def _wid():
    return lax.axis_index("c") * NS + lax.axis_index("s")


@functools.partial(
    pl.kernel,
    out_type=jax.ShapeDtypeStruct((ROWS_PAD * 128,), jnp.float32),
    mesh=_MESH,
    name="sc_count",
    compiler_params=pltpu.CompilerParams(needs_layout_passes=False),
    scratch_types=[
        pltpu.VMEM((C1,), jnp.int32),
        pltpu.VMEM((RPT * 128,), jnp.float32),
    ],
)
def _count_sc(key_hbm, cnt_hbm, key_v, cnt_t):
    wid = _wid()
    key_lo = wid * RPT * 128

    def _zero(i, _):
        cnt_t[pl.ds(i * 16, 16)] = jnp.zeros((16,), jnp.float32)
        return 0
    lax.fori_loop(0, RPT * 8, _zero, 0)

    ones = jnp.ones((16,), jnp.float32)
    def _chunk(ch, _):
        pltpu.sync_copy(key_hbm.at[pl.ds(ch * C1, C1)], key_v)
        def _grp(g, _):
            kv = key_v[pl.ds(g * 16, 16)]
            lkey = kv - key_lo
            msk = (kv >= key_lo) & (kv < key_lo + RPT * 128)
            plsc.addupdate_scatter(cnt_t, [lkey], ones, mask=msk)
            return 0
        lax.fori_loop(0, C1 // 16, _grp, 0)
        return 0
    lax.fori_loop(0, E_PAD // C1, _chunk, 0)

    pltpu.sync_copy(cnt_t, cnt_hbm.at[pl.ds(key_lo, RPT * 128)])


CN = 192
NCHUNK_N = 56

@functools.partial(
    pl.kernel,
    out_type=(jax.ShapeDtypeStruct((E_PAD, D), jnp.float32),
              jax.ShapeDtypeStruct((E_PAD,), jnp.float32)),
    mesh=_MESH,
    name="sc_gather_norm",
    compiler_params=pltpu.CompilerParams(needs_layout_passes=False),
    scratch_types=[
        pltpu.VMEM((CN,), jnp.int32), pltpu.VMEM((CN,), jnp.int32),
        pltpu.VMEM((CN,), jnp.int32), pltpu.VMEM((CN,), jnp.int32),
        pltpu.VMEM((CN,), jnp.int32),
        pltpu.VMEM((CN, D), jnp.float32), pltpu.VMEM((CN, D), jnp.float32),
        pltpu.VMEM((CN, 128), jnp.float32), pltpu.VMEM((CN, 128), jnp.float32),
        pltpu.VMEM((CN,), jnp.float32),
        pltpu.SemaphoreType.DMA, pltpu.SemaphoreType.DMA,
        pltpu.SemaphoreType.DMA, pltpu.SemaphoreType.DMA,
    ],
)
def _gather_norm_sc(tab_hbm, cnt_hbm, src_hbm, dst_hbm, rel_hbm,
                    out_hbm, norm_hbm,
                    is0, is1, id0, id1, rel_v, xr0, xr1, cr0, cr1, nrm_v,
                    sx0, sx1, sc0, sc1):
    base = _wid() * CN * NCHUNK_N
    lanes = lax.iota(jnp.int32, 16)

    def _start(off, is_v, id_v, xr_v, cr_v, semx, semc):
        pltpu.sync_copy(src_hbm.at[pl.ds(off, CN)], is_v)
        pltpu.async_copy(tab_hbm.at[is_v], xr_v, semx)
        pltpu.sync_copy(dst_hbm.at[pl.ds(off, CN)], id_v)
        pltpu.async_copy(cnt_hbm.at[id_v], cr_v, semc)

    def _drain(off, is_v, id_v, xr_v, cr_v, semx, semc):
        pltpu.make_async_copy(tab_hbm.at[is_v], xr_v, semx).wait()
        pltpu.sync_copy(xr_v, out_hbm.at[pl.ds(off, CN)])
        pltpu.make_async_copy(cnt_hbm.at[id_v], cr_v, semc).wait()
        pltpu.sync_copy(rel_hbm.at[pl.ds(off, CN)], rel_v)
        def _grp(g, _):
            rows = g * 16 + lanes
            lane = rel_v[pl.ds(g * 16, 16)]
            cv = plsc.load_gather(cr_v, [rows, lane])
            nrm_v[pl.ds(g * 16, 16)] = 1.0 / jnp.maximum(cv, 1.0)
            return 0
        lax.fori_loop(0, CN // 16, _grp, 0)
        pltpu.sync_copy(nrm_v, norm_hbm.at[pl.ds(off, CN)])

    _start(pl.multiple_of(base, CN), is0, id0, xr0, cr0, sx0, sc0)
    def _body(jj, _):
        offa = pl.multiple_of(base + (2 * jj) * CN, CN)
        offb = pl.multiple_of(base + (2 * jj + 1) * CN, CN)
        _start(offb, is1, id1, xr1, cr1, sx1, sc1)
        _drain(offa, is0, id0, xr0, cr0, sx0, sc0)
        @pl.when(2 * jj + 2 < NCHUNK_N)
        def _():
            offc = pl.multiple_of(base + (2 * jj + 2) * CN, CN)
            _start(offc, is0, id0, xr0, cr0, sx0, sc0)
        _drain(offb, is1, id1, xr1, cr1, sx1, sc1)
        return 0
    lax.fori_loop(0, NCHUNK_N // 2, _body, 0)


CG = 192
NCHUNK_G = 56

@functools.partial(
    pl.kernel,
    out_type=jax.ShapeDtypeStruct((E_PAD, D), jnp.float32),
    mesh=_MESH,
    name="sc_gather",
    compiler_params=pltpu.CompilerParams(needs_layout_passes=False),
    scratch_types=[
        pltpu.VMEM((CG,), jnp.int32), pltpu.VMEM((CG,), jnp.int32),
        pltpu.VMEM((CG,), jnp.int32), pltpu.VMEM((CG,), jnp.int32),
        pltpu.VMEM((CG, D), jnp.float32), pltpu.VMEM((CG, D), jnp.float32),
        pltpu.VMEM((CG, D), jnp.float32), pltpu.VMEM((CG, D), jnp.float32),
        pltpu.SemaphoreType.DMA, pltpu.SemaphoreType.DMA,
        pltpu.SemaphoreType.DMA, pltpu.SemaphoreType.DMA,
    ],
)
def _gather_sc(tab_hbm, idx_hbm, out_hbm, i0, i1, i2, i3, r0, r1, r2, r3,
               s0, s1, s2, s3):
    base = _wid() * CG * NCHUNK_G
    ib = [i0, i1, i2, i3]
    rb = [r0, r1, r2, r3]
    sb = [s0, s1, s2, s3]

    def _start(g, b):
        off = pl.multiple_of(base + g * CG, CG)
        pltpu.sync_copy(idx_hbm.at[pl.ds(off, CG)], ib[b])
        pltpu.async_copy(tab_hbm.at[ib[b]], rb[b], sb[b])

    def _drain(g, b):
        off = pl.multiple_of(base + g * CG, CG)
        pltpu.make_async_copy(tab_hbm.at[ib[b]], rb[b], sb[b]).wait()
        pltpu.sync_copy(rb[b], out_hbm.at[pl.ds(off, CG)])

    for b in range(3):
        _start(b, b)
    def _body(kk, _):
        for b in range(4):
            g = kk * 4 + b
            @pl.when(g + 3 < NCHUNK_G)
            def _():
                _start(g + 3, (b + 3) % 4)
            _drain(g, b)
        return 0
    lax.fori_loop(0, NCHUNK_G // 4, _body, 0)


@functools.partial(
    pl.kernel,
    out_type=jax.ShapeDtypeStruct((NC, N, D), jnp.float32),
    mesh=_MESH,
    name="sc_scatter",
    compiler_params=pltpu.CompilerParams(needs_layout_passes=False),
    scratch_types=[
        pltpu.VMEM((C,), jnp.int32),
        pltpu.VMEM((C, D), jnp.float32),
        pltpu.VMEM_SHARED((N, D), jnp.float32),
    ],
)
def _scatter_sc(m_hbm, dst_hbm, out_hbm, idx_v, row_v, acc_sh):
    cid = lax.axis_index("c")
    sid = lax.axis_index("s")

    def _zrow(i, _):
        row_v[i // 8, pl.ds((i % 8) * 16, 16)] = jnp.zeros((16,), jnp.float32)
        return 0
    lax.fori_loop(0, C * 8, _zrow, 0)
    slab = sid * 640

    @pl.when(sid < 15)
    def _():
        pltpu.sync_copy(row_v.at[pl.ds(0, 320)], acc_sh.at[pl.ds(slab, 320)])
        pltpu.sync_copy(row_v.at[pl.ds(0, 320)], acc_sh.at[pl.ds(slab + 320, 320)])

    @pl.when(sid == 15)
    def _():
        pltpu.sync_copy(row_v.at[pl.ds(0, 200)], acc_sh.at[pl.ds(slab, 200)])
        pltpu.sync_copy(row_v.at[pl.ds(0, 200)], acc_sh.at[pl.ds(slab + 200, 200)])

    plsc.subcore_barrier()

    per_tile = E_PAD // NC // NS
    base = cid * (E_PAD // NC) + sid * per_tile
    def _body(i, _):
        off = pl.multiple_of(base + i * C, C)
        pltpu.sync_copy(dst_hbm.at[pl.ds(off, C)], idx_v)
        pltpu.sync_copy(m_hbm.at[pl.ds(off, C)], row_v)
        pltpu.sync_copy(row_v, acc_sh.at[idx_v], add=True)
        return 0
    lax.fori_loop(0, per_tile // C, _body, 0)

    plsc.subcore_barrier()

    @pl.when(sid < 15)
    def _():
        pltpu.sync_copy(acc_sh.at[pl.ds(slab, 640)], out_hbm.at[cid, pl.ds(slab, 640)])

    @pl.when(sid == 15)
    def _():
        pltpu.sync_copy(acc_sh.at[pl.ds(slab, 400)], out_hbm.at[cid, pl.ds(slab, 400)])


C2 = 128
K2 = 40
E_PAD2 = NW * C2 * K2

@functools.partial(
    pl.kernel,
    out_type=(jax.ShapeDtypeStruct((E_PAD2,), jnp.float32),
              jax.ShapeDtypeStruct((E_PAD2,), jnp.float32)),
    mesh=_MESH,
    name="sc_decode",
    compiler_params=pltpu.CompilerParams(needs_layout_passes=False),
    scratch_types=[
        pltpu.VMEM((R, D), jnp.float32),
        pltpu.VMEM((C2,), jnp.int32), pltpu.VMEM((C2,), jnp.int32),
        pltpu.VMEM((C2,), jnp.int32), pltpu.VMEM((C2,), jnp.int32),
        pltpu.VMEM((C2,), jnp.int32), pltpu.VMEM((C2,), jnp.int32),
        pltpu.VMEM((C2,), jnp.int32),
        pltpu.VMEM((C2, D), jnp.float32), pltpu.VMEM((C2, D), jnp.float32),
        pltpu.VMEM((C2, D), jnp.float32), pltpu.VMEM((C2, D), jnp.float32),
        pltpu.VMEM((C2, D), jnp.float32), pltpu.VMEM((C2, D), jnp.float32),
        pltpu.VMEM((C2 * 16,), jnp.float32),
        pltpu.VMEM((C2 * 16,), jnp.float32),
        pltpu.VMEM((C2,), jnp.float32),
        pltpu.VMEM((C2,), jnp.float32),
        pltpu.SemaphoreType.DMA, pltpu.SemaphoreType.DMA,
        pltpu.SemaphoreType.DMA, pltpu.SemaphoreType.DMA,
        pltpu.SemaphoreType.DMA, pltpu.SemaphoreType.DMA,
    ],
)
def _decode_sc(x_hbm, remb_hbm, head_hbm, tail_hbm, ntail_hbm, rel_hbm,
               pos_hbm, neg_hbm,
               re_t, ih0, ih1, it0, it1, in0, in1, ir_v,
               xh0, xh1, xt0, xt1, xn0, xn1,
               ap_v, an_v, pb_v, nb_v, sh0, sh1, st0, st1, sn0, sn1):
    base = _wid() * C2 * K2
    lanes = lax.iota(jnp.int32, 16)
    pltpu.sync_copy(remb_hbm, re_t)
    ihb, itb, inb = [ih0, ih1], [it0, it1], [in0, in1]
    xhb, xtb, xnb = [xh0, xh1], [xt0, xt1], [xn0, xn1]
    shb, stb, snb = [sh0, sh1], [st0, st1], [sn0, sn1]

    def _start(g, b):
        off = pl.multiple_of(base + g * C2, 8)
        pltpu.sync_copy(head_hbm.at[pl.ds(off, C2)], ihb[b])
        pltpu.async_copy(x_hbm.at[ihb[b]], xhb[b], shb[b])
        pltpu.sync_copy(tail_hbm.at[pl.ds(off, C2)], itb[b])
        pltpu.async_copy(x_hbm.at[itb[b]], xtb[b], stb[b])
        pltpu.sync_copy(ntail_hbm.at[pl.ds(off, C2)], inb[b])
        pltpu.async_copy(x_hbm.at[inb[b]], xnb[b], snb[b])

    def _drain(g, b):
        off = pl.multiple_of(base + g * C2, 8)
        xh_v, xt_v, xn_v = xhb[b], xtb[b], xnb[b]
        pltpu.make_async_copy(x_hbm.at[ihb[b]], xh_v, shb[b]).wait()
        pltpu.make_async_copy(x_hbm.at[itb[b]], xt_v, stb[b]).wait()
        pltpu.make_async_copy(x_hbm.at[inb[b]], xn_v, snb[b]).wait()
        pltpu.sync_copy(rel_hbm.at[pl.ds(off, C2)], ir_v)

        def _row16(g1, _):
            rv = ir_v[pl.ds(g1 * 16, 16)]
            for j in range(16):
                r = g1 * 16 + j
                rr = rv[j]
                accp = jnp.zeros((16,), jnp.float32)
                accn = jnp.zeros((16,), jnp.float32)
                for k in range(D // 16):
                    sl = pl.ds(k * 16, 16)
                    h = xh_v[r, sl] * re_t[rr, sl]
                    accp = accp + h * xt_v[r, sl]
                    accn = accn + h * xn_v[r, sl]
                ap_v[pl.ds(r * 16, 16)] = accp
                an_v[pl.ds(r * 16, 16)] = accn
            return 0
        lax.fori_loop(0, C2 // 16, _row16, 0)

        def _red(g2, _):
            idx0 = (g2 * 16 + lanes) * 16
            ps = jnp.zeros((16,), jnp.float32)
            ns = jnp.zeros((16,), jnp.float32)
            for j in range(16):
                ps = ps + plsc.load_gather(ap_v, [idx0 + j])
                ns = ns + plsc.load_gather(an_v, [idx0 + j])
            pb_v[pl.ds(g2 * 16, 16)] = ps
            nb_v[pl.ds(g2 * 16, 16)] = ns
            return 0
        lax.fori_loop(0, C2 // 16, _red, 0)

        pltpu.sync_copy(pb_v, pos_hbm.at[pl.ds(off, C2)])
        pltpu.sync_copy(nb_v, neg_hbm.at[pl.ds(off, C2)])

    _start(0, 0)
    def _body(jj, _):
        ja = 2 * jj
        jb = 2 * jj + 1
        _start(jb, 1)
        _drain(ja, 0)
        @pl.when(jb + 1 < K2)
        def _():
            _start(jb + 1, 0)
        _drain(jb, 1)
        return 0
    lax.fori_loop(0, K2 // 2, _body, 0)


CM = 2048


def _basis_body(x_ref, n_ref, rel_ref, comb_ref, bas_ref, o_ref):
    xn = x_ref[...] * n_ref[...]
    rel = rel_ref[...]
    onehot = (lax.broadcasted_iota(jnp.int32, (CM, 128), 1) == rel
              ).astype(jnp.float32)
    coef = jnp.dot(onehot, comb_ref[...],
                   preferred_element_type=jnp.float32)
    xb = jnp.concatenate([xn * coef[:, b][:, None] for b in range(B)], axis=1)
    o_ref[...] = jnp.dot(xb, bas_ref[...], preferred_element_type=jnp.float32)


def _basis_tc(xsrc, norm2d, rel2d, comb_p, bases_r):
    return pl.pallas_call(
        _basis_body,
        grid=(E_PAD // CM,),
        in_specs=[
            pl.BlockSpec((CM, D), lambda i: (i, 0)),
            pl.BlockSpec((CM, 1), lambda i: (i, 0)),
            pl.BlockSpec((CM, 1), lambda i: (i, 0)),
            pl.BlockSpec((128, B), lambda i: (0, 0)),
            pl.BlockSpec((B * D, H), lambda i: (0, 0)),
        ],
        out_specs=pl.BlockSpec((CM, H), lambda i: (i, 0)),
        out_shape=jax.ShapeDtypeStruct((E_PAD, H), jnp.float32),
    )(xsrc, norm2d, rel2d, comb_p, bases_r)


def _merge_body(relu, p_ref, b_ref, o_ref):
    s = p_ref[0] + p_ref[1] + b_ref[...]
    o_ref[...] = jnp.maximum(s, 0.0) if relu else s


def _merge_tc(partial, bias2d, relu):
    return pl.pallas_call(
        functools.partial(_merge_body, relu),
        grid=(10,),
        in_specs=[
            pl.BlockSpec((NC, 1000, D), lambda i: (0, i, 0)),
            pl.BlockSpec((1, D), lambda i: (0, 0)),
        ],
        out_specs=pl.BlockSpec((1000, D), lambda i: (i, 0)),
        out_shape=jax.ShapeDtypeStruct((N, D), jnp.float32),
    )(partial, bias2d)


def _x0_body(e_ref, b_ref, o_ref):
    o_ref[...] = jnp.maximum(e_ref[...] + b_ref[...], 0.0)


def _x0_tc(emb, ebias):
    return pl.pallas_call(
        _x0_body,
        grid=(10,),
        in_specs=[
            pl.BlockSpec((1000, D), lambda i: (i, 0)),
            pl.BlockSpec((1, D), lambda i: (0, 0)),
        ],
        out_specs=pl.BlockSpec((1000, D), lambda i: (i, 0)),
        out_shape=jax.ShapeDtypeStruct((N, D), jnp.float32),
    )(emb, ebias)


def kernel(entity_embedding, entity_embedding_bias, bases1, comb1, b1,
           bases2, comb2, b2, rel_emb, edge_index, edge_type):
    i32 = jnp.int32
    src = edge_index[0].astype(i32)
    dst = edge_index[1].astype(i32)
    rel = edge_type.astype(i32)
    loop = jnp.arange(N, dtype=i32)

    npad = E_PAD - E_F
    src_f = jnp.concatenate([src, dst, loop, jnp.zeros((npad,), i32)])
    dst_f = jnp.concatenate([dst, src, loop, jnp.zeros((npad,), i32)])
    rel_f = jnp.concatenate([rel, rel + R, jnp.full((N,), 2 * R, i32),
                             jnp.full((npad,), NUM_REL, i32)])
    key_f = dst_f * 128 + rel_f

    comb1p = jnp.zeros((128, B), jnp.float32).at[:NUM_REL].set(comb1)
    comb2p = jnp.zeros((128, B), jnp.float32).at[:NUM_REL].set(comb2)

    cnt = _count_sc(key_f).reshape(ROWS_PAD, 128)

    x = _x0_tc(entity_embedding, entity_embedding_bias)

    rel2d = rel_f.reshape(E_PAD, 1)
    bases1_r = bases1.reshape(B * D, H)
    bases2_r = bases2.reshape(B * H, D)

    xsrc, norm = _gather_norm_sc(x, cnt, src_f, dst_f, rel_f)
    norm2d = norm.reshape(E_PAD, 1)
    m = _basis_tc(xsrc, norm2d, rel2d, comb1p, bases1_r)
    partial = _scatter_sc(m, dst_f)
    x = _merge_tc(partial, b1.reshape(1, H), relu=True)

    xsrc = _gather_sc(x, src_f)
    m = _basis_tc(xsrc, norm2d, rel2d, comb2p, bases2_r)
    partial = _scatter_sc(m, dst_f)
    x = _merge_tc(partial, b2.reshape(1, D), relu=False)

    neg_tail = jax.random.randint(jax.random.key(42), (E,), 0, N).astype(i32)
    dpad = E_PAD2 - E
    zpad = jnp.zeros((dpad,), i32)
    pos, neg = _decode_sc(x, rel_emb,
                          jnp.concatenate([src, zpad]),
                          jnp.concatenate([dst, zpad]),
                          jnp.concatenate([neg_tail, zpad]),
                          jnp.concatenate([rel, zpad]))
    pos, neg = pos[:E], neg[:E]

    loss = (jnp.sum(jax.nn.softplus(-pos)) + jnp.sum(jax.nn.softplus(neg))) / (2.0 * E)
    flags = jnp.concatenate([jnp.zeros((E,), jnp.int32), jnp.ones((E,), jnp.int32)])
    _, sflags = lax.sort((jnp.concatenate([pos, neg]), flags), num_keys=2)
    nb = jnp.cumsum(sflags) - sflags
    per_pos = jnp.where(sflags == 0, nb, 0)
    hi = jnp.sum(per_pos >> 12)
    lo = jnp.sum(per_pos & 0xFFF)
    auc = (hi.astype(jnp.float32) * 4096.0 + lo.astype(jnp.float32)) / (float(E) * float(E))
    return (pos, loss, auc)


# speedup vs baseline: 6.6780x; 1.0322x over previous
import functools

import jax
import jax.numpy as jnp
from jax import lax
from jax.experimental import pallas as pl
from jax.experimental.pallas import tpu as pltpu
from jax.experimental.pallas import tpu_sc as plsc

N = 10000
R = 50
NUM_REL = 2 * R + 1
D = 128
H = 128
B = 8
E = 160000
E_F = 2 * E + N

NC = 2
NS = 16
NW = NC * NS
C = 384
NCHUNK = 28
E_PAD = NW * C * NCHUNK

ROWS_PAD = 10240
RPT = ROWS_PAD // NW
C1 = 2048

_MESH = plsc.VectorSubcoreMesh(core_axis_name="c", subcore_axis_name="s",
                               num_cores=NC, num_subcores=NS)


def _wid():
    return lax.axis_index("c") * NS + lax.axis_index("s")


@functools.partial(
    pl.kernel,
    out_type=jax.ShapeDtypeStruct((ROWS_PAD * 128,), jnp.float32),
    mesh=_MESH,
    name="sc_count",
    compiler_params=pltpu.CompilerParams(needs_layout_passes=False),
    scratch_types=[
        pltpu.VMEM((C1,), jnp.int32),
        pltpu.VMEM((RPT * 128,), jnp.float32),
    ],
)
def _count_sc(key_hbm, cnt_hbm, key_v, cnt_t):
    wid = _wid()
    key_lo = wid * RPT * 128

    def _zero(i, _):
        cnt_t[pl.ds(i * 16, 16)] = jnp.zeros((16,), jnp.float32)
        return 0
    lax.fori_loop(0, RPT * 8, _zero, 0)

    ones = jnp.ones((16,), jnp.float32)
    def _chunk(ch, _):
        pltpu.sync_copy(key_hbm.at[pl.ds(ch * C1, C1)], key_v)
        def _grp(g, _):
            kv = key_v[pl.ds(g * 16, 16)]
            lkey = kv - key_lo
            msk = (kv >= key_lo) & (kv < key_lo + RPT * 128)
            plsc.addupdate_scatter(cnt_t, [lkey], ones, mask=msk)
            return 0
        lax.fori_loop(0, C1 // 16, _grp, 0)
        return 0
    lax.fori_loop(0, E_PAD // C1, _chunk, 0)

    pltpu.sync_copy(cnt_t, cnt_hbm.at[pl.ds(key_lo, RPT * 128)])


CN = 192
NCHUNK_N = 56

@functools.partial(
    pl.kernel,
    out_type=(jax.ShapeDtypeStruct((E_PAD, D), jnp.float32),
              jax.ShapeDtypeStruct((E_PAD,), jnp.float32)),
    mesh=_MESH,
    name="sc_gather_norm",
    compiler_params=pltpu.CompilerParams(needs_layout_passes=False),
    scratch_types=[
        pltpu.VMEM((CN,), jnp.int32), pltpu.VMEM((CN,), jnp.int32),
        pltpu.VMEM((CN,), jnp.int32), pltpu.VMEM((CN,), jnp.int32),
        pltpu.VMEM((CN,), jnp.int32),
        pltpu.VMEM((CN, D), jnp.float32), pltpu.VMEM((CN, D), jnp.float32),
        pltpu.VMEM((CN, 128), jnp.float32), pltpu.VMEM((CN, 128), jnp.float32),
        pltpu.VMEM((CN,), jnp.float32),
        pltpu.SemaphoreType.DMA, pltpu.SemaphoreType.DMA,
        pltpu.SemaphoreType.DMA, pltpu.SemaphoreType.DMA,
    ],
)
def _gather_norm_sc(tab_hbm, cnt_hbm, src_hbm, dst_hbm, rel_hbm,
                    out_hbm, norm_hbm,
                    is0, is1, id0, id1, rel_v, xr0, xr1, cr0, cr1, nrm_v,
                    sx0, sx1, sc0, sc1):
    base = _wid() * CN * NCHUNK_N
    lanes = lax.iota(jnp.int32, 16)

    def _start(off, is_v, id_v, xr_v, cr_v, semx, semc):
        pltpu.sync_copy(src_hbm.at[pl.ds(off, CN)], is_v)
        pltpu.async_copy(tab_hbm.at[is_v], xr_v, semx)
        pltpu.sync_copy(dst_hbm.at[pl.ds(off, CN)], id_v)
        pltpu.async_copy(cnt_hbm.at[id_v], cr_v, semc)

    def _drain(off, is_v, id_v, xr_v, cr_v, semx, semc):
        pltpu.make_async_copy(tab_hbm.at[is_v], xr_v, semx).wait()
        pltpu.sync_copy(xr_v, out_hbm.at[pl.ds(off, CN)])
        pltpu.make_async_copy(cnt_hbm.at[id_v], cr_v, semc).wait()
        pltpu.sync_copy(rel_hbm.at[pl.ds(off, CN)], rel_v)
        def _grp(g, _):
            rows = g * 16 + lanes
            lane = rel_v[pl.ds(g * 16, 16)]
            cv = plsc.load_gather(cr_v, [rows, lane])
            nrm_v[pl.ds(g * 16, 16)] = 1.0 / jnp.maximum(cv, 1.0)
            return 0
        lax.fori_loop(0, CN // 16, _grp, 0)
        pltpu.sync_copy(nrm_v, norm_hbm.at[pl.ds(off, CN)])

    _start(pl.multiple_of(base, CN), is0, id0, xr0, cr0, sx0, sc0)
    def _body(jj, _):
        offa = pl.multiple_of(base + (2 * jj) * CN, CN)
        offb = pl.multiple_of(base + (2 * jj + 1) * CN, CN)
        _start(offb, is1, id1, xr1, cr1, sx1, sc1)
        _drain(offa, is0, id0, xr0, cr0, sx0, sc0)
        @pl.when(2 * jj + 2 < NCHUNK_N)
        def _():
            offc = pl.multiple_of(base + (2 * jj + 2) * CN, CN)
            _start(offc, is0, id0, xr0, cr0, sx0, sc0)
        _drain(offb, is1, id1, xr1, cr1, sx1, sc1)
        return 0
    lax.fori_loop(0, NCHUNK_N // 2, _body, 0)


CG = 192
NCHUNK_G = 56

@functools.partial(
    pl.kernel,
    out_type=jax.ShapeDtypeStruct((E_PAD, D), jnp.float32),
    mesh=_MESH,
    name="sc_gather",
    compiler_params=pltpu.CompilerParams(needs_layout_passes=False),
    scratch_types=[
        pltpu.VMEM((CG,), jnp.int32), pltpu.VMEM((CG,), jnp.int32),
        pltpu.VMEM((CG,), jnp.int32), pltpu.VMEM((CG,), jnp.int32),
        pltpu.VMEM((CG, D), jnp.float32), pltpu.VMEM((CG, D), jnp.float32),
        pltpu.VMEM((CG, D), jnp.float32), pltpu.VMEM((CG, D), jnp.float32),
        pltpu.SemaphoreType.DMA, pltpu.SemaphoreType.DMA,
        pltpu.SemaphoreType.DMA, pltpu.SemaphoreType.DMA,
        pltpu.SemaphoreType.DMA, pltpu.SemaphoreType.DMA,
        pltpu.SemaphoreType.DMA, pltpu.SemaphoreType.DMA,
    ],
)
def _gather_sc(tab_hbm, idx_hbm, out_hbm, i0, i1, i2, i3, r0, r1, r2, r3,
               s0, s1, s2, s3, s4, s5, s6, s7):
    base = _wid() * CG * NCHUNK_G
    ib = [i0, i1, i2, i3]
    rb = [r0, r1, r2, r3]
    sa = [s0, s1, s2, s3]
    sc = [s4, s5, s6, s7]
    HG = CG // 2

    def _start(g, b):
        off = pl.multiple_of(base + g * CG, CG)
        pltpu.sync_copy(idx_hbm.at[pl.ds(off, CG)], ib[b])
        pltpu.async_copy(tab_hbm.at[ib[b].at[pl.ds(0, HG)]],
                         rb[b].at[pl.ds(0, HG)], sa[b])
        pltpu.async_copy(tab_hbm.at[ib[b].at[pl.ds(HG, HG)]],
                         rb[b].at[pl.ds(HG, HG)], sc[b])

    def _drain(g, b):
        off = pl.multiple_of(base + g * CG, CG)
        pltpu.make_async_copy(tab_hbm.at[ib[b].at[pl.ds(0, HG)]],
                              rb[b].at[pl.ds(0, HG)], sa[b]).wait()
        pltpu.make_async_copy(tab_hbm.at[ib[b].at[pl.ds(HG, HG)]],
                              rb[b].at[pl.ds(HG, HG)], sc[b]).wait()
        pltpu.sync_copy(rb[b], out_hbm.at[pl.ds(off, CG)])

    for b in range(3):
        _start(b, b)
    def _body(kk, _):
        for b in range(4):
            g = kk * 4 + b
            @pl.when(g + 3 < NCHUNK_G)
            def _():
                _start(g + 3, (b + 3) % 4)
            _drain(g, b)
        return 0
    lax.fori_loop(0, NCHUNK_G // 4, _body, 0)


@functools.partial(
    pl.kernel,
    out_type=jax.ShapeDtypeStruct((NC, N, D), jnp.float32),
    mesh=_MESH,
    name="sc_scatter",
    compiler_params=pltpu.CompilerParams(needs_layout_passes=False),
    scratch_types=[
        pltpu.VMEM((C,), jnp.int32),
        pltpu.VMEM((C, D), jnp.float32),
        pltpu.VMEM_SHARED((N, D), jnp.float32),
    ],
)
def _scatter_sc(m_hbm, dst_hbm, out_hbm, idx_v, row_v, acc_sh):
    cid = lax.axis_index("c")
    sid = lax.axis_index("s")

    def _zrow(i, _):
        row_v[i // 8, pl.ds((i % 8) * 16, 16)] = jnp.zeros((16,), jnp.float32)
        return 0
    lax.fori_loop(0, C * 8, _zrow, 0)
    slab = sid * 640

    @pl.when(sid < 15)
    def _():
        pltpu.sync_copy(row_v.at[pl.ds(0, 320)], acc_sh.at[pl.ds(slab, 320)])
        pltpu.sync_copy(row_v.at[pl.ds(0, 320)], acc_sh.at[pl.ds(slab + 320, 320)])

    @pl.when(sid == 15)
    def _():
        pltpu.sync_copy(row_v.at[pl.ds(0, 200)], acc_sh.at[pl.ds(slab, 200)])
        pltpu.sync_copy(row_v.at[pl.ds(0, 200)], acc_sh.at[pl.ds(slab + 200, 200)])

    plsc.subcore_barrier()

    per_tile = E_PAD // NC // NS
    base = cid * (E_PAD // NC) + sid * per_tile
    def _body(i, _):
        off = pl.multiple_of(base + i * C, C)
        pltpu.sync_copy(dst_hbm.at[pl.ds(off, C)], idx_v)
        pltpu.sync_copy(m_hbm.at[pl.ds(off, C)], row_v)
        pltpu.sync_copy(row_v, acc_sh.at[idx_v], add=True)
        return 0
    lax.fori_loop(0, per_tile // C, _body, 0)

    plsc.subcore_barrier()

    @pl.when(sid < 15)
    def _():
        pltpu.sync_copy(acc_sh.at[pl.ds(slab, 640)], out_hbm.at[cid, pl.ds(slab, 640)])

    @pl.when(sid == 15)
    def _():
        pltpu.sync_copy(acc_sh.at[pl.ds(slab, 400)], out_hbm.at[cid, pl.ds(slab, 400)])


C2 = 128
K2 = 40
E_PAD2 = NW * C2 * K2

@functools.partial(
    pl.kernel,
    out_type=(jax.ShapeDtypeStruct((E_PAD2,), jnp.float32),
              jax.ShapeDtypeStruct((E_PAD2,), jnp.float32)),
    mesh=_MESH,
    name="sc_decode",
    compiler_params=pltpu.CompilerParams(needs_layout_passes=False),
    scratch_types=[
        pltpu.VMEM((R, D), jnp.float32),
        pltpu.VMEM((C2,), jnp.int32), pltpu.VMEM((C2,), jnp.int32),
        pltpu.VMEM((C2,), jnp.int32), pltpu.VMEM((C2,), jnp.int32),
        pltpu.VMEM((C2,), jnp.int32), pltpu.VMEM((C2,), jnp.int32),
        pltpu.VMEM((C2,), jnp.int32),
        pltpu.VMEM((C2, D), jnp.float32), pltpu.VMEM((C2, D), jnp.float32),
        pltpu.VMEM((C2, D), jnp.float32), pltpu.VMEM((C2, D), jnp.float32),
        pltpu.VMEM((C2, D), jnp.float32), pltpu.VMEM((C2, D), jnp.float32),
        pltpu.VMEM((C2 * 16,), jnp.float32),
        pltpu.VMEM((C2 * 16,), jnp.float32),
        pltpu.VMEM((C2,), jnp.float32),
        pltpu.VMEM((C2,), jnp.float32),
        pltpu.SemaphoreType.DMA, pltpu.SemaphoreType.DMA,
        pltpu.SemaphoreType.DMA, pltpu.SemaphoreType.DMA,
        pltpu.SemaphoreType.DMA, pltpu.SemaphoreType.DMA,
    ],
)
def _decode_sc(x_hbm, remb_hbm, head_hbm, tail_hbm, ntail_hbm, rel_hbm,
               pos_hbm, neg_hbm,
               re_t, ih0, ih1, it0, it1, in0, in1, ir_v,
               xh0, xh1, xt0, xt1, xn0, xn1,
               ap_v, an_v, pb_v, nb_v, sh0, sh1, st0, st1, sn0, sn1):
    base = _wid() * C2 * K2
    lanes = lax.iota(jnp.int32, 16)
    pltpu.sync_copy(remb_hbm, re_t)
    ihb, itb, inb = [ih0, ih1], [it0, it1], [in0, in1]
    xhb, xtb, xnb = [xh0, xh1], [xt0, xt1], [xn0, xn1]
    shb, stb, snb = [sh0, sh1], [st0, st1], [sn0, sn1]

    def _start(g, b):
        off = pl.multiple_of(base + g * C2, 8)
        pltpu.sync_copy(head_hbm.at[pl.ds(off, C2)], ihb[b])
        pltpu.async_copy(x_hbm.at[ihb[b]], xhb[b], shb[b])
        pltpu.sync_copy(tail_hbm.at[pl.ds(off, C2)], itb[b])
        pltpu.async_copy(x_hbm.at[itb[b]], xtb[b], stb[b])
        pltpu.sync_copy(ntail_hbm.at[pl.ds(off, C2)], inb[b])
        pltpu.async_copy(x_hbm.at[inb[b]], xnb[b], snb[b])

    def _drain(g, b):
        off = pl.multiple_of(base + g * C2, 8)
        xh_v, xt_v, xn_v = xhb[b], xtb[b], xnb[b]
        pltpu.make_async_copy(x_hbm.at[ihb[b]], xh_v, shb[b]).wait()
        pltpu.make_async_copy(x_hbm.at[itb[b]], xt_v, stb[b]).wait()
        pltpu.make_async_copy(x_hbm.at[inb[b]], xn_v, snb[b]).wait()
        pltpu.sync_copy(rel_hbm.at[pl.ds(off, C2)], ir_v)

        def _row16(g1, _):
            rv = ir_v[pl.ds(g1 * 16, 16)]
            for j in range(16):
                r = g1 * 16 + j
                rr = rv[j]
                accp = jnp.zeros((16,), jnp.float32)
                accn = jnp.zeros((16,), jnp.float32)
                for k in range(D // 16):
                    sl = pl.ds(k * 16, 16)
                    h = xh_v[r, sl] * re_t[rr, sl]
                    accp = accp + h * xt_v[r, sl]
                    accn = accn + h * xn_v[r, sl]
                ap_v[pl.ds(r * 16, 16)] = accp
                an_v[pl.ds(r * 16, 16)] = accn
            return 0
        lax.fori_loop(0, C2 // 16, _row16, 0)

        def _red(g2, _):
            idx0 = (g2 * 16 + lanes) * 16
            ps = jnp.zeros((16,), jnp.float32)
            ns = jnp.zeros((16,), jnp.float32)
            for j in range(16):
                ps = ps + plsc.load_gather(ap_v, [idx0 + j])
                ns = ns + plsc.load_gather(an_v, [idx0 + j])
            pb_v[pl.ds(g2 * 16, 16)] = ps
            nb_v[pl.ds(g2 * 16, 16)] = ns
            return 0
        lax.fori_loop(0, C2 // 16, _red, 0)

        pltpu.sync_copy(pb_v, pos_hbm.at[pl.ds(off, C2)])
        pltpu.sync_copy(nb_v, neg_hbm.at[pl.ds(off, C2)])

    _start(0, 0)
    def _body(jj, _):
        ja = 2 * jj
        jb = 2 * jj + 1
        _start(jb, 1)
        _drain(ja, 0)
        @pl.when(jb + 1 < K2)
        def _():
            _start(jb + 1, 0)
        _drain(jb, 1)
        return 0
    lax.fori_loop(0, K2 // 2, _body, 0)


CM = 2048


def _basis_body(x_ref, n_ref, rel_ref, comb_ref, bas_ref, o_ref):
    xn = x_ref[...] * n_ref[...]
    rel = rel_ref[...]
    onehot = (lax.broadcasted_iota(jnp.int32, (CM, 128), 1) == rel
              ).astype(jnp.float32)
    coef = jnp.dot(onehot, comb_ref[...],
                   preferred_element_type=jnp.float32)
    t = jnp.dot(xn, bas_ref[...], preferred_element_type=jnp.float32)
    acc = t[:, 0:H] * coef[:, 0][:, None]
    for b in range(1, B):
        acc = acc + t[:, b * H:(b + 1) * H] * coef[:, b][:, None]
    o_ref[...] = acc


def _basis_tc(xsrc, norm2d, rel2d, comb_p, bases_r):
    return pl.pallas_call(
        _basis_body,
        grid=(E_PAD // CM,),
        in_specs=[
            pl.BlockSpec((CM, D), lambda i: (i, 0)),
            pl.BlockSpec((CM, 1), lambda i: (i, 0)),
            pl.BlockSpec((CM, 1), lambda i: (i, 0)),
            pl.BlockSpec((128, B), lambda i: (0, 0)),
            pl.BlockSpec((D, B * H), lambda i: (0, 0)),
        ],
        out_specs=pl.BlockSpec((CM, H), lambda i: (i, 0)),
        out_shape=jax.ShapeDtypeStruct((E_PAD, H), jnp.float32),
    )(xsrc, norm2d, rel2d, comb_p, bases_r)


def _merge_body(relu, p_ref, b_ref, o_ref):
    s = p_ref[0] + p_ref[1] + b_ref[...]
    o_ref[...] = jnp.maximum(s, 0.0) if relu else s


def _merge_tc(partial, bias2d, relu):
    return pl.pallas_call(
        functools.partial(_merge_body, relu),
        grid=(10,),
        in_specs=[
            pl.BlockSpec((NC, 1000, D), lambda i: (0, i, 0)),
            pl.BlockSpec((1, D), lambda i: (0, 0)),
        ],
        out_specs=pl.BlockSpec((1000, D), lambda i: (i, 0)),
        out_shape=jax.ShapeDtypeStruct((N, D), jnp.float32),
    )(partial, bias2d)


def _x0_body(e_ref, b_ref, o_ref):
    o_ref[...] = jnp.maximum(e_ref[...] + b_ref[...], 0.0)


def _x0_tc(emb, ebias):
    return pl.pallas_call(
        _x0_body,
        grid=(10,),
        in_specs=[
            pl.BlockSpec((1000, D), lambda i: (i, 0)),
            pl.BlockSpec((1, D), lambda i: (0, 0)),
        ],
        out_specs=pl.BlockSpec((1000, D), lambda i: (i, 0)),
        out_shape=jax.ShapeDtypeStruct((N, D), jnp.float32),
    )(emb, ebias)


def kernel(entity_embedding, entity_embedding_bias, bases1, comb1, b1,
           bases2, comb2, b2, rel_emb, edge_index, edge_type):
    i32 = jnp.int32
    src = edge_index[0].astype(i32)
    dst = edge_index[1].astype(i32)
    rel = edge_type.astype(i32)
    loop = jnp.arange(N, dtype=i32)

    npad = E_PAD - E_F
    src_f = jnp.concatenate([src, dst, loop, jnp.zeros((npad,), i32)])
    dst_f = jnp.concatenate([dst, src, loop, jnp.zeros((npad,), i32)])
    rel_f = jnp.concatenate([rel, rel + R, jnp.full((N,), 2 * R, i32),
                             jnp.full((npad,), NUM_REL, i32)])
    key_f = dst_f * 128 + rel_f

    comb1p = jnp.zeros((128, B), jnp.float32).at[:NUM_REL].set(comb1)
    comb2p = jnp.zeros((128, B), jnp.float32).at[:NUM_REL].set(comb2)

    cnt = _count_sc(key_f).reshape(ROWS_PAD, 128)

    x = _x0_tc(entity_embedding, entity_embedding_bias)

    rel2d = rel_f.reshape(E_PAD, 1)
    bases1_r = jnp.transpose(bases1, (1, 0, 2)).reshape(D, B * H)
    bases2_r = jnp.transpose(bases2, (1, 0, 2)).reshape(H, B * D)

    xsrc, norm = _gather_norm_sc(x, cnt, src_f, dst_f, rel_f)
    norm2d = norm.reshape(E_PAD, 1)
    m = _basis_tc(xsrc, norm2d, rel2d, comb1p, bases1_r)
    partial = _scatter_sc(m, dst_f)
    x = _merge_tc(partial, b1.reshape(1, H), relu=True)

    xsrc = _gather_sc(x, src_f)
    m = _basis_tc(xsrc, norm2d, rel2d, comb2p, bases2_r)
    partial = _scatter_sc(m, dst_f)
    x = _merge_tc(partial, b2.reshape(1, D), relu=False)

    neg_tail = jax.random.randint(jax.random.key(42), (E,), 0, N).astype(i32)
    dpad = E_PAD2 - E
    zpad = jnp.zeros((dpad,), i32)
    pos, neg = _decode_sc(x, rel_emb,
                          jnp.concatenate([src, zpad]),
                          jnp.concatenate([dst, zpad]),
                          jnp.concatenate([neg_tail, zpad]),
                          jnp.concatenate([rel, zpad]))
    pos, neg = pos[:E], neg[:E]

    loss = (jnp.sum(jax.nn.softplus(-pos)) + jnp.sum(jax.nn.softplus(neg))) / (2.0 * E)
    flags = jnp.concatenate([jnp.zeros((E,), jnp.int32), jnp.ones((E,), jnp.int32)])
    _, sflags = lax.sort((jnp.concatenate([pos, neg]), flags), num_keys=2)
    nb = jnp.cumsum(sflags) - sflags
    per_pos = jnp.where(sflags == 0, nb, 0)
    hi = jnp.sum(per_pos >> 12)
    lo = jnp.sum(per_pos & 0xFFF)
    auc = (hi.astype(jnp.float32) * 4096.0 + lo.astype(jnp.float32)) / (float(E) * float(E))
    return (pos, loss, auc)
